# baseline (speedup 1.0000x reference)
import functools

import jax
import jax.numpy as jnp
from jax.experimental import pallas as pl
from jax.experimental.pallas import tpu as pltpu

_NC = 80
_B = 3
_LBL_C = _NC + 5
_PRD_C = _NC + 5 * _B
_CH = 4
_NCHUNK = 256 // _CH


def _sqrt_scale(x):
    return jnp.sign(x) * jnp.sqrt(jnp.abs(x))


def _chunk_losses(lbl4, prd4):
    n = _CH * 28 * 28
    lbl = lbl4.reshape(n, _LBL_C)
    prd = prd4.reshape(n, _PRD_C)

    lq = jnp.transpose(lbl[:, _NC:_NC + 5])
    pq = jnp.transpose(prd[:, _NC:_NC + 5 * _B])

    conf = lq[0:1, :]
    lx, ly, lw, lh = lq[1:2, :], lq[2:3, :], lq[3:4, :], lq[4:5, :]
    pc = [pq[5 * j + 0:5 * j + 1, :] for j in range(_B)]
    px = [pq[5 * j + 1:5 * j + 2, :] for j in range(_B)]
    py = [pq[5 * j + 2:5 * j + 3, :] for j in range(_B)]
    pw = [pq[5 * j + 3:5 * j + 4, :] for j in range(_B)]
    ph = [pq[5 * j + 4:5 * j + 5, :] for j in range(_B)]

    mask_obj = (conf > 0.5).astype(jnp.float32)
    mask_no = (conf != 1.0).astype(jnp.float32)

    def iou(j):
        ax1, ax2 = lx - lw * 0.5, lx + lw * 0.5
        ay1, ay2 = ly - lh * 0.5, ly + lh * 0.5
        bx1, bx2 = px[j] - pw[j] * 0.5, px[j] + pw[j] * 0.5
        by1, by2 = py[j] - ph[j] * 0.5, py[j] + ph[j] * 0.5
        iw = jnp.maximum(jnp.minimum(ax2, bx2) - jnp.maximum(ax1, bx1), 0.0)
        ih = jnp.maximum(jnp.minimum(ay2, by2) - jnp.maximum(ay1, by1), 0.0)
        inter = iw * ih
        union = lw * lh + pw[j] * ph[j] - inter + 1e-6
        return inter / union

    ious = [iou(j) for j in range(_B)]
    best_i, bc, bx, by, bw, bh = ious[0], pc[0], px[0], py[0], pw[0], ph[0]
    for j in range(1, _B):
        keep = best_i >= ious[j]
        best_i = jnp.where(keep, best_i, ious[j])
        bc = jnp.where(keep, bc, pc[j])
        bx = jnp.where(keep, bx, px[j])
        by = jnp.where(keep, by, py[j])
        bw = jnp.where(keep, bw, pw[j])
        bh = jnp.where(keep, bh, ph[j])

    loc = jnp.sum(mask_obj * ((lx - bx) ** 2 + (ly - by) ** 2))
    size = jnp.sum(mask_obj * ((_sqrt_scale(lw) - _sqrt_scale(bw)) ** 2
                               + (_sqrt_scale(lh) - _sqrt_scale(bh)) ** 2))
    pobj = jnp.sum(mask_obj * (conf - bc) ** 2)
    pno = jnp.sum(mask_no * ((conf - pc[0]) ** 2 + (conf - pc[1]) ** 2
                             + (conf - pc[2]) ** 2))

    mask_obj_col = (lbl[:, _NC:_NC + 1] > 0.5).astype(jnp.float32)
    d = lbl - prd[:, :_LBL_C]
    per_lane = jax.lax.dot_general(
        d * d, mask_obj_col, (((0,), (0,)), ((), ())),
        preferred_element_type=jnp.float32)
    lane = jax.lax.broadcasted_iota(jnp.int32, (_LBL_C, 1), 0)
    cls = jnp.sum(jnp.where(lane < _NC, per_lane, 0.0))

    return loc, size, pobj, pno, cls


_NSLOT = 4


def _body(lbl_hbm, prd_hbm, out_ref,
          lbuf0, lbuf1, lbuf2, lbuf3,
          pbuf0, pbuf1, pbuf2, pbuf3,
          sem0, sem1, sem2, sem3):
    lbufs = (lbuf0, lbuf1, lbuf2, lbuf3)
    pbufs = (pbuf0, pbuf1, pbuf2, pbuf3)
    sems = (sem0, sem1, sem2, sem3)

    def start(g, s):
        pltpu.make_async_copy(
            lbl_hbm.at[pl.ds(g * _CH, _CH)], lbufs[s], sems[s].at[0]).start()
        pltpu.make_async_copy(
            prd_hbm.at[pl.ds(g * _CH, _CH)], pbufs[s], sems[s].at[1]).start()

    def wait(g, s):
        pltpu.make_async_copy(
            lbl_hbm.at[pl.ds(g * _CH, _CH)], lbufs[s], sems[s].at[0]).wait()
        pltpu.make_async_copy(
            prd_hbm.at[pl.ds(g * _CH, _CH)], pbufs[s], sems[s].at[1]).wait()

    for s in range(_NSLOT):
        start(s, s)

    def quad_body(t, acc):
        g0 = _NSLOT * t
        for s in range(_NSLOT):
            wait(g0 + s, s)
            p = _chunk_losses(lbufs[s][...], pbufs[s][...])
            acc = tuple(a + q for a, q in zip(acc, p))

            @pl.when(t < _NCHUNK // _NSLOT - 1)
            def _():
                start(g0 + s + _NSLOT, s)

        return acc

    acc = jax.lax.fori_loop(
        0, _NCHUNK // _NSLOT, quad_body,
        tuple(jnp.float32(0.0) for _ in range(5)))

    m = 256 * 28 * 28
    s_mb = 1.0 / (m + _B)
    s_mc = 1.0 / (m + _NC)
    scaled = (acc[0] * s_mb, acc[1] * s_mb, acc[2] * s_mb,
              acc[3] * s_mb, acc[4] * s_mc)
    lane2 = jax.lax.broadcasted_iota(jnp.int32, (8, 128), 1)
    v = ((lane2 == 0) * scaled[0] + (lane2 == 1) * scaled[1]
         + (lane2 == 2) * scaled[2] + (lane2 == 3) * scaled[3]
         + (lane2 == 4) * scaled[4])
    out_ref[...] = v.astype(jnp.float32)


@functools.partial(jax.jit, static_argnames=("interpret",))
def _run(label, pred, interpret=False):
    out = pl.pallas_call(
        _body,
        in_specs=[
            pl.BlockSpec(memory_space=pl.ANY),
            pl.BlockSpec(memory_space=pl.ANY),
        ],
        out_specs=pl.BlockSpec(memory_space=pltpu.VMEM),
        out_shape=jax.ShapeDtypeStruct((8, 128), jnp.float32),
        scratch_shapes=(
            [pltpu.VMEM((_CH, 28, 28, _LBL_C), jnp.float32)] * 4
            + [pltpu.VMEM((_CH, 28, 28, _PRD_C), jnp.float32)] * 4
            + [pltpu.SemaphoreType.DMA((2,))] * 4),
        interpret=interpret,
    )(label, pred)
    return (out[0, 0], out[0, 1], out[0, 2], out[0, 3], out[0, 4])


def kernel(label, pred):
    return _run(label, pred)


# speedup vs baseline: 1.1648x; 1.1648x over previous
import functools

import jax
import jax.numpy as jnp
from jax.experimental import pallas as pl
from jax.experimental.pallas import tpu as pltpu

_NC = 80
_B = 3
_LBL_C = _NC + 5
_PRD_C = _NC + 5 * _B


def _iou_rows(lx, ly, lw, lh, px, py, pw, ph):
    ax1, ax2 = lx - lw * 0.5, lx + lw * 0.5
    ay1, ay2 = ly - lh * 0.5, ly + lh * 0.5
    bx1, bx2 = px - pw * 0.5, px + pw * 0.5
    by1, by2 = py - ph * 0.5, py + ph * 0.5
    iw = jnp.maximum(jnp.minimum(ax2, bx2) - jnp.maximum(ax1, bx1), 0.0)
    ih = jnp.maximum(jnp.minimum(ay2, by2) - jnp.maximum(ay1, by1), 0.0)
    inter = iw * ih
    union = lw * lh + pw * ph - inter + 1e-6
    return inter / union


def _sqrt_scale(x):
    return jnp.sign(x) * jnp.sqrt(jnp.abs(x))


def _body(lbl_ref, prd_ref, out_ref):
    i = pl.program_id(0)
    sb = lbl_ref.shape[0]
    lbl = lbl_ref[...].reshape(sb * 28, _LBL_C)
    prd = prd_ref[...].reshape(sb * 28, _PRD_C)

    lq = jnp.transpose(lbl[:, _NC:_NC + 5])
    pq = jnp.transpose(prd[:, _NC:_NC + 5 * _B])

    conf = lq[0:1, :]
    lx, ly, lw, lh = lq[1:2, :], lq[2:3, :], lq[3:4, :], lq[4:5, :]
    pc = [pq[5 * j + 0:5 * j + 1, :] for j in range(_B)]
    px = [pq[5 * j + 1:5 * j + 2, :] for j in range(_B)]
    py = [pq[5 * j + 2:5 * j + 3, :] for j in range(_B)]
    pw = [pq[5 * j + 3:5 * j + 4, :] for j in range(_B)]
    ph = [pq[5 * j + 4:5 * j + 5, :] for j in range(_B)]

    mask_obj = (conf > 0.5).astype(jnp.float32)
    mask_no = (conf != 1.0).astype(jnp.float32)

    ious = [_iou_rows(lx, ly, lw, lh, px[j], py[j], pw[j], ph[j])
            for j in range(_B)]

    best_i, bc, bx, by, bw, bh = ious[0], pc[0], px[0], py[0], pw[0], ph[0]
    for j in range(1, _B):
        keep = best_i >= ious[j]
        best_i = jnp.where(keep, best_i, ious[j])
        bc = jnp.where(keep, bc, pc[j])
        bx = jnp.where(keep, bx, px[j])
        by = jnp.where(keep, by, py[j])
        bw = jnp.where(keep, bw, pw[j])
        bh = jnp.where(keep, bh, ph[j])

    loc = jnp.sum(mask_obj * ((lx - bx) ** 2 + (ly - by) ** 2))
    size = jnp.sum(mask_obj * ((_sqrt_scale(lw) - _sqrt_scale(bw)) ** 2
                               + (_sqrt_scale(lh) - _sqrt_scale(bh)) ** 2))
    pobj = jnp.sum(mask_obj * (conf - bc) ** 2)
    pno = jnp.sum(mask_no * ((conf - pc[0]) ** 2 + (conf - pc[1]) ** 2
                             + (conf - pc[2]) ** 2))

    mask_obj_col = (lbl[:, _NC:_NC + 1] > 0.5).astype(jnp.float32)
    lane = jax.lax.broadcasted_iota(jnp.int32, (1, _LBL_C), 1)
    lane_mask = (lane < _NC).astype(jnp.float32)
    d = lbl - prd[:, :_LBL_C]
    cls = jnp.sum(d * d * (mask_obj_col * lane_mask))

    @pl.when(i == 0)
    def _init():
        for k in range(5):
            out_ref[k] = 0.0

    out_ref[0] += loc
    out_ref[1] += size
    out_ref[2] += pobj
    out_ref[3] += pno
    out_ref[4] += cls

    @pl.when(i == pl.num_programs(0) - 1)
    def _scale():
        m = 256 * 28 * 28
        s_mb = 1.0 / (m + _B)
        s_mc = 1.0 / (m + _NC)
        out_ref[0] = out_ref[0] * s_mb
        out_ref[1] = out_ref[1] * s_mb
        out_ref[2] = out_ref[2] * s_mb
        out_ref[3] = out_ref[3] * s_mb
        out_ref[4] = out_ref[4] * s_mc


@functools.partial(jax.jit, static_argnames=("interpret",))
def _run(label, pred, interpret=False):
    nslab = label.shape[0] * label.shape[1]
    lbl3 = label.reshape(nslab, label.shape[2], _LBL_C)
    prd3 = pred.reshape(nslab, pred.shape[2], _PRD_C)
    sb = 256
    grid = nslab // sb
    out = pl.pallas_call(
        _body,
        grid=(grid,),
        in_specs=[
            pl.BlockSpec((sb, 28, _LBL_C), lambda i: (i, 0, 0)),
            pl.BlockSpec((sb, 28, _PRD_C), lambda i: (i, 0, 0)),
        ],
        out_specs=pl.BlockSpec(memory_space=pltpu.SMEM),
        out_shape=jax.ShapeDtypeStruct((5,), jnp.float32),
        interpret=interpret,
    )(lbl3, prd3)
    return (out[0], out[1], out[2], out[3], out[4])


def kernel(label, pred):
    return _run(label, pred)
